# Initial kernel scaffold; baseline (speedup 1.0000x reference)
import jax
import jax.numpy as jnp
from jax.experimental import pallas as pl


def kernel(xyz, params):
    raise NotImplementedError("write your pallas kernel here")


# speedup vs baseline: 9.5884x; 9.5884x over previous
import functools

import jax
import jax.numpy as jnp
import numpy as np
from jax.experimental import pallas as pl
from jax.experimental.pallas import tpu as pltpu

_B = 8
_N = 2048
_K = 32
_NPOINTS = (1024, 512, 256, 128)
_RADII = (0.05, 0.1, 0.2, 0.3)
_BN_EPS = 1e-5


def _fps_body(n, s, xyz_ref, out_ref):
    xs = xyz_ref[:, 0, :]
    ys = xyz_ref[:, 1, :]
    zs = xyz_ref[:, 2, :]
    iota = jax.lax.broadcasted_iota(jnp.int32, (_B, n), 1)
    iota_s = jax.lax.broadcasted_iota(jnp.int32, (_B, s), 1)

    def body(i, carry):
        dist, far, cxo, cyo, czo = carry
        oh = (iota == far).astype(jnp.float32)
        cx = jnp.sum(xs * oh, axis=-1, keepdims=True)
        cy = jnp.sum(ys * oh, axis=-1, keepdims=True)
        cz = jnp.sum(zs * oh, axis=-1, keepdims=True)
        msel = iota_s == i
        cxo = jnp.where(msel, cx, cxo)
        cyo = jnp.where(msel, cy, cyo)
        czo = jnp.where(msel, cz, czo)
        dx = xs - cx
        dy = ys - cy
        dz = zs - cz
        dd = (dx * dx + dy * dy) + dz * dz
        dist = jnp.minimum(dist, dd)
        mx = jnp.max(dist, axis=-1, keepdims=True)
        far = jnp.min(jnp.where(dist == mx, iota, n), axis=-1, keepdims=True)
        return dist, far.astype(jnp.int32), cxo, cyo, czo

    dist0 = jnp.full((_B, n), 1e10, dtype=jnp.float32)
    far0 = jnp.zeros((_B, 1), dtype=jnp.int32)
    czero = jnp.zeros((_B, s), dtype=jnp.float32)
    _, _, cxo, cyo, czo = jax.lax.fori_loop(
        0, s, body, (dist0, far0, czero, czero, czero))
    out_ref[:, 0, :] = cxo
    out_ref[:, 1, :] = cyo
    out_ref[:, 2, :] = czo


def _fps(xyz_c, s):
    b, _, n = xyz_c.shape
    return pl.pallas_call(
        functools.partial(_fps_body, n, s),
        out_shape=jax.ShapeDtypeStruct((b, 3, s), jnp.float32),
    )(xyz_c)


def _group_body(n, sblk, c3, r2, xyzT_ref, qp_ref, p_ref, out_ref):
    xt = xyzT_ref[0]
    qp = qp_ref[0]
    q = qp[0:3, :]
    p = p_ref[0]
    p3 = p[0:3, :]
    xsq = jnp.sum(xt * xt, axis=1, keepdims=True)
    qsq = jnp.sum(q * q, axis=0, keepdims=True)
    d = -2.0 * jnp.dot(xt.astype(jnp.bfloat16), q.astype(jnp.bfloat16),
                       preferred_element_type=jnp.float32)
    d = d + qsq
    d = d + xsq
    mask = d <= r2
    mi = mask.astype(jnp.int32)
    c = mi
    sh = 1
    while sh < n:
        shifted = jnp.concatenate(
            [jnp.zeros((sh, sblk), jnp.int32), c[: n - sh, :]], axis=0)
        c = c + shifted
        sh *= 2
    count = c[n - 1 : n, :]
    slot = jnp.where(mask, c - 1, -1)
    row = jax.lax.broadcasted_iota(jnp.int32, (n, sblk), 0)
    slot = jnp.where((row == n - 1) & (count == 0), 0, slot)
    for k in range(_K):
        tgt = jnp.where(count > k, k, 0)
        sel = (slot == tgt).astype(jnp.float32)
        gxyz = jax.lax.dot_general(
            p3, sel, (((1,), (0,)), ((), ())),
            precision=jax.lax.Precision.HIGHEST,
            preferred_element_type=jnp.float32)
        gfeat = jnp.dot(p[3:, :], sel,
                        preferred_element_type=jnp.float32)
        out_ref[0, 0:3, k, :] = gxyz - q
        out_ref[0, 3:, k, :] = gfeat


def _group(xyz_c, pts, new_xyz, radius):
    b, _, n = xyz_c.shape
    s = new_xyz.shape[2]
    c_feat = pts.shape[1]
    c3 = c_feat + 3
    sblk = min(s, 256)
    p = jnp.concatenate([xyz_c, pts], axis=1)
    xyzT = jnp.transpose(xyz_c, (0, 2, 1))
    qpad = jnp.concatenate(
        [new_xyz, jnp.zeros((b, c_feat, s), jnp.float32)], axis=1)
    grid = (b, s // sblk)
    out = pl.pallas_call(
        functools.partial(_group_body, n, sblk, c3, radius * radius),
        grid=grid,
        in_specs=[
            pl.BlockSpec((1, n, 3), lambda i, j: (i, 0, 0)),
            pl.BlockSpec((1, c3, sblk), lambda i, j: (i, 0, j)),
            pl.BlockSpec((1, c3, n), lambda i, j: (i, 0, 0)),
        ],
        out_specs=pl.BlockSpec((1, c3, _K, sblk), lambda i, j: (i, 0, 0, j)),
        out_shape=jax.ShapeDtypeStruct((b, c3, _K, s), jnp.float32),
    )(xyzT, qpad, p)
    return out.reshape(b, c3, _K * s)


def _bn_in(y, stats, inv_m):
    ysum, ysq, gamma, beta = stats
    mean = ysum[...] * inv_m
    var = ysq[...] * inv_m - mean * mean
    x = (y - mean) / jnp.sqrt(var + _BN_EPS)
    x = x * gamma[...] + beta[...]
    return jnp.maximum(x, 0.0)


def _conv1_body(x_ref, w_ref, b_ref, y_ref, s_ref, q_ref):
    y = jnp.dot(w_ref[...].astype(jnp.bfloat16),
                x_ref[0].astype(jnp.bfloat16),
                preferred_element_type=jnp.float32) + b_ref[...]
    y_ref[0] = y
    ms = jnp.sum(y, axis=1, keepdims=True)
    ss = jnp.sum(y * y, axis=1, keepdims=True)

    @pl.when(pl.program_id(0) == 0)
    def _():
        s_ref[...] = ms
        q_ref[...] = ss

    @pl.when(pl.program_id(0) != 0)
    def _():
        s_ref[...] = s_ref[...] + ms
        q_ref[...] = q_ref[...] + ss


def _conv_mid_body(inv_m, yp_ref, sum_ref, sq_ref, g_ref, bt_ref,
                   w_ref, b_ref, y_ref, s_ref, q_ref):
    x = _bn_in(yp_ref[0], (sum_ref, sq_ref, g_ref, bt_ref), inv_m)
    y = jnp.dot(w_ref[...].astype(jnp.bfloat16), x.astype(jnp.bfloat16),
                preferred_element_type=jnp.float32) + b_ref[...]
    y_ref[0] = y
    ms = jnp.sum(y, axis=1, keepdims=True)
    ss = jnp.sum(y * y, axis=1, keepdims=True)

    @pl.when(pl.program_id(0) == 0)
    def _():
        s_ref[...] = ms
        q_ref[...] = ss

    @pl.when(pl.program_id(0) != 0)
    def _():
        s_ref[...] = s_ref[...] + ms
        q_ref[...] = q_ref[...] + ss


def _pool_body(inv_m, s, yp_ref, sum_ref, sq_ref, g_ref, bt_ref, out_ref):
    x = _bn_in(yp_ref[0], (sum_ref, sq_ref, g_ref, bt_ref), inv_m)
    acc = x[:, 0:s]
    for k in range(1, _K):
        acc = jnp.maximum(acc, x[:, k * s : (k + 1) * s])
    out_ref[0] = acc


def _stats_specs(c):
    return [
        pl.BlockSpec((c, 1), lambda i: (0, 0)),
        pl.BlockSpec((c, 1), lambda i: (0, 0)),
    ]


def _conv1(x, w, bias):
    b, ci, m = x.shape
    co = w.shape[0]
    return pl.pallas_call(
        _conv1_body,
        grid=(b,),
        in_specs=[
            pl.BlockSpec((1, ci, m), lambda i: (i, 0, 0)),
            pl.BlockSpec((co, ci), lambda i: (0, 0)),
            pl.BlockSpec((co, 1), lambda i: (0, 0)),
        ],
        out_specs=[pl.BlockSpec((1, co, m), lambda i: (i, 0, 0))]
        + _stats_specs(co),
        out_shape=[
            jax.ShapeDtypeStruct((b, co, m), jnp.float32),
            jax.ShapeDtypeStruct((co, 1), jnp.float32),
            jax.ShapeDtypeStruct((co, 1), jnp.float32),
        ],
    )(x, w, bias.reshape(co, 1))


def _conv_mid(yp, stats, gamma, beta, w, bias, m_count):
    b, ci, m = yp.shape
    co = w.shape[0]
    ysum, ysq = stats
    return pl.pallas_call(
        functools.partial(_conv_mid_body, 1.0 / m_count),
        grid=(b,),
        in_specs=[
            pl.BlockSpec((1, ci, m), lambda i: (i, 0, 0)),
        ] + _stats_specs(ci) + _stats_specs(ci) + [
            pl.BlockSpec((co, ci), lambda i: (0, 0)),
            pl.BlockSpec((co, 1), lambda i: (0, 0)),
        ],
        out_specs=[pl.BlockSpec((1, co, m), lambda i: (i, 0, 0))]
        + _stats_specs(co),
        out_shape=[
            jax.ShapeDtypeStruct((b, co, m), jnp.float32),
            jax.ShapeDtypeStruct((co, 1), jnp.float32),
            jax.ShapeDtypeStruct((co, 1), jnp.float32),
        ],
    )(yp, ysum, ysq, gamma.reshape(ci, 1), beta.reshape(ci, 1),
      w, bias.reshape(co, 1))


def _pool(yp, stats, gamma, beta, s, m_count):
    b, c, m = yp.shape
    ysum, ysq = stats
    return pl.pallas_call(
        functools.partial(_pool_body, 1.0 / m_count, s),
        grid=(b,),
        in_specs=[
            pl.BlockSpec((1, c, m), lambda i: (i, 0, 0)),
        ] + _stats_specs(c) + _stats_specs(c),
        out_specs=pl.BlockSpec((1, c, s), lambda i: (i, 0, 0)),
        out_shape=jax.ShapeDtypeStruct((b, c, s), jnp.float32),
    )(yp, ysum, ysq, gamma.reshape(c, 1), beta.reshape(c, 1))


@jax.jit
def kernel(xyz, params):
    l_xyz = [xyz[:, :3, :]]
    l_points = [xyz]
    cur_xyz = xyz[:, :3, :]
    cur_pts = xyz
    for i in range(4):
        s = _NPOINTS[i]
        new_xyz = _fps(cur_xyz, s)
        x = _group(cur_xyz, cur_pts, new_xyz, _RADII[i])
        m_count = _B * _K * s
        layer = params[i]
        w, bias, gamma, beta = layer[0]
        y, ysum, ysq = _conv1(x, w, bias)
        for j in range(1, len(layer)):
            w2, bias2, g2, b2 = layer[j]
            y, ysum2, ysq2 = _conv_mid(
                y, (ysum, ysq), gamma, beta, w2, bias2, m_count)
            ysum, ysq = ysum2, ysq2
            gamma, beta = g2, b2
        out = _pool(y, (ysum, ysq), gamma, beta, s, m_count)
        l_xyz.append(new_xyz)
        l_points.append(out)
        cur_xyz = new_xyz
        cur_pts = out
    return tuple(l_xyz) + tuple(l_points)


# speedup vs baseline: 9.7124x; 1.0129x over previous
import functools

import jax
import jax.numpy as jnp
import numpy as np
from jax.experimental import pallas as pl
from jax.experimental.pallas import tpu as pltpu

_B = 8
_N = 2048
_K = 32
_NPOINTS = (1024, 512, 256, 128)
_RADII = (0.05, 0.1, 0.2, 0.3)
_BN_EPS = 1e-5


def _fps_body(n, s, xyz_ref, out_ref):
    xs = xyz_ref[:, 0, :]
    ys = xyz_ref[:, 1, :]
    zs = xyz_ref[:, 2, :]
    iota = jax.lax.broadcasted_iota(jnp.int32, (_B, n), 1)
    iota_s = jax.lax.broadcasted_iota(jnp.int32, (_B, s), 1)

    def body(i, carry):
        dist, far, cxo, cyo, czo = carry
        oh = (iota == far).astype(jnp.float32)
        cx = jnp.sum(xs * oh, axis=-1, keepdims=True)
        cy = jnp.sum(ys * oh, axis=-1, keepdims=True)
        cz = jnp.sum(zs * oh, axis=-1, keepdims=True)
        msel = iota_s == i
        cxo = jnp.where(msel, cx, cxo)
        cyo = jnp.where(msel, cy, cyo)
        czo = jnp.where(msel, cz, czo)
        dx = xs - cx
        dy = ys - cy
        dz = zs - cz
        dd = (dx * dx + dy * dy) + dz * dz
        dist = jnp.minimum(dist, dd)
        mx = jnp.max(dist, axis=-1, keepdims=True)
        far = jnp.min(jnp.where(dist == mx, iota, n), axis=-1, keepdims=True)
        return dist, far.astype(jnp.int32), cxo, cyo, czo

    dist0 = jnp.full((_B, n), 1e10, dtype=jnp.float32)
    far0 = jnp.zeros((_B, 1), dtype=jnp.int32)
    czero = jnp.zeros((_B, s), dtype=jnp.float32)
    _, _, cxo, cyo, czo = jax.lax.fori_loop(
        0, s, body, (dist0, far0, czero, czero, czero))
    out_ref[:, 0, :] = cxo
    out_ref[:, 1, :] = cyo
    out_ref[:, 2, :] = czo


def _fps(xyz_c, s):
    b, _, n = xyz_c.shape
    return pl.pallas_call(
        functools.partial(_fps_body, n, s),
        out_shape=jax.ShapeDtypeStruct((b, 3, s), jnp.float32),
    )(xyz_c)


_CCH = 256


def _group_body(n, sblk, c3, r2, xyzT_ref, qp_ref, pb_ref, p3_ref, out_ref):
    xt = xyzT_ref[0]
    qp = qp_ref[0]
    q = qp[0:3, :]
    pb = pb_ref[0]
    xsq = jnp.sum(xt * xt, axis=1, keepdims=True)
    qsq = jnp.sum(q * q, axis=0, keepdims=True)
    d = -2.0 * jnp.dot(xt.astype(jnp.bfloat16), q.astype(jnp.bfloat16),
                       preferred_element_type=jnp.float32)
    d = d + qsq
    d = d + xsq
    mask = d <= r2
    mi = jnp.where(mask, 1.0, 0.0).astype(jnp.bfloat16)
    ra = jax.lax.broadcasted_iota(jnp.int32, (_CCH, _CCH), 0)
    ca = jax.lax.broadcasted_iota(jnp.int32, (_CCH, _CCH), 1)
    ltri = jnp.where(ra >= ca, 1.0, 0.0).astype(jnp.bfloat16)
    chunks = []
    carry = jnp.zeros((1, sblk), jnp.float32)
    for j in range(n // _CCH):
        seg = mi[j * _CCH : (j + 1) * _CCH, :]
        cs = jnp.dot(ltri, seg, preferred_element_type=jnp.float32)
        chunks.append(cs + carry)
        carry = carry + cs[_CCH - 1 : _CCH, :]
    c = jnp.concatenate(chunks, axis=0)
    count = carry
    slot = jnp.where(mask, c - 1.0, -1.0)
    row = jax.lax.broadcasted_iota(jnp.int32, (n, sblk), 0)
    slot = jnp.where((row == n - 1) & (count == 0.0), 0.0, slot)
    slotc = jnp.minimum(slot, 33.0)
    p3 = p3_ref[0]
    for k in range(_K):
        tgt = jnp.where(count > k, float(k), 0.0)
        self32 = jnp.where(slotc == tgt, 1.0, 0.0)
        sel = self32.astype(jnp.bfloat16)
        g = jnp.dot(pb, sel, preferred_element_type=jnp.float32)
        gxyz = jnp.dot(p3, self32, precision=jax.lax.Precision.HIGHEST,
                       preferred_element_type=jnp.float32)
        out_ref[0, 0:3, k, :] = gxyz - q
        out_ref[0, 3:, k, :] = g[6:, :]


def _group(xyz_c, pts, new_xyz, radius):
    b, _, n = xyz_c.shape
    s = new_xyz.shape[2]
    c_feat = pts.shape[1]
    c3 = c_feat + 3
    sblk = min(s, 256)
    xyz_hi = xyz_c.astype(jnp.bfloat16)
    xyz_lo = (xyz_c - xyz_hi.astype(jnp.float32)).astype(jnp.bfloat16)
    pb = jnp.concatenate([xyz_hi, xyz_lo, pts.astype(jnp.bfloat16)], axis=1)
    xyzT = jnp.transpose(xyz_c, (0, 2, 1))
    qpad = jnp.concatenate(
        [new_xyz, jnp.zeros((b, c_feat, s), jnp.float32)], axis=1)
    grid = (b, s // sblk)
    out = pl.pallas_call(
        functools.partial(_group_body, n, sblk, c3, radius * radius),
        grid=grid,
        in_specs=[
            pl.BlockSpec((1, n, 3), lambda i, j: (i, 0, 0)),
            pl.BlockSpec((1, c3, sblk), lambda i, j: (i, 0, j)),
            pl.BlockSpec((1, c3 + 3, n), lambda i, j: (i, 0, 0)),
            pl.BlockSpec((1, 3, n), lambda i, j: (i, 0, 0)),
        ],
        out_specs=pl.BlockSpec((1, c3, _K, sblk), lambda i, j: (i, 0, 0, j)),
        out_shape=jax.ShapeDtypeStruct((b, c3, _K, s), jnp.float32),
    )(xyzT, qpad, pb, xyz_c)
    return out.reshape(b, c3, _K * s)


def _bn_in(y, stats, inv_m):
    ysum, ysq, gamma, beta = stats
    mean = ysum[...] * inv_m
    var = ysq[...] * inv_m - mean * mean
    x = (y - mean) / jnp.sqrt(var + _BN_EPS)
    x = x * gamma[...] + beta[...]
    return jnp.maximum(x, 0.0)


def _conv1_body(x_ref, w_ref, b_ref, y_ref, s_ref, q_ref):
    y = jnp.dot(w_ref[...].astype(jnp.bfloat16),
                x_ref[0].astype(jnp.bfloat16),
                preferred_element_type=jnp.float32) + b_ref[...]
    y_ref[0] = y
    ms = jnp.sum(y, axis=1, keepdims=True)
    ss = jnp.sum(y * y, axis=1, keepdims=True)

    @pl.when(pl.program_id(0) == 0)
    def _():
        s_ref[...] = ms
        q_ref[...] = ss

    @pl.when(pl.program_id(0) != 0)
    def _():
        s_ref[...] = s_ref[...] + ms
        q_ref[...] = q_ref[...] + ss


def _conv_mid_body(inv_m, yp_ref, sum_ref, sq_ref, g_ref, bt_ref,
                   w_ref, b_ref, y_ref, s_ref, q_ref):
    x = _bn_in(yp_ref[0], (sum_ref, sq_ref, g_ref, bt_ref), inv_m)
    y = jnp.dot(w_ref[...].astype(jnp.bfloat16), x.astype(jnp.bfloat16),
                preferred_element_type=jnp.float32) + b_ref[...]
    y_ref[0] = y
    ms = jnp.sum(y, axis=1, keepdims=True)
    ss = jnp.sum(y * y, axis=1, keepdims=True)

    @pl.when(pl.program_id(0) == 0)
    def _():
        s_ref[...] = ms
        q_ref[...] = ss

    @pl.when(pl.program_id(0) != 0)
    def _():
        s_ref[...] = s_ref[...] + ms
        q_ref[...] = q_ref[...] + ss


def _pool_body(inv_m, s, yp_ref, sum_ref, sq_ref, g_ref, bt_ref, out_ref):
    x = _bn_in(yp_ref[0], (sum_ref, sq_ref, g_ref, bt_ref), inv_m)
    acc = x[:, 0:s]
    for k in range(1, _K):
        acc = jnp.maximum(acc, x[:, k * s : (k + 1) * s])
    out_ref[0] = acc


def _stats_specs(c):
    return [
        pl.BlockSpec((c, 1), lambda i: (0, 0)),
        pl.BlockSpec((c, 1), lambda i: (0, 0)),
    ]


def _conv1(x, w, bias):
    b, ci, m = x.shape
    co = w.shape[0]
    return pl.pallas_call(
        _conv1_body,
        grid=(b,),
        in_specs=[
            pl.BlockSpec((1, ci, m), lambda i: (i, 0, 0)),
            pl.BlockSpec((co, ci), lambda i: (0, 0)),
            pl.BlockSpec((co, 1), lambda i: (0, 0)),
        ],
        out_specs=[pl.BlockSpec((1, co, m), lambda i: (i, 0, 0))]
        + _stats_specs(co),
        out_shape=[
            jax.ShapeDtypeStruct((b, co, m), jnp.float32),
            jax.ShapeDtypeStruct((co, 1), jnp.float32),
            jax.ShapeDtypeStruct((co, 1), jnp.float32),
        ],
    )(x, w, bias.reshape(co, 1))


def _conv_mid(yp, stats, gamma, beta, w, bias, m_count):
    b, ci, m = yp.shape
    co = w.shape[0]
    ysum, ysq = stats
    return pl.pallas_call(
        functools.partial(_conv_mid_body, 1.0 / m_count),
        grid=(b,),
        in_specs=[
            pl.BlockSpec((1, ci, m), lambda i: (i, 0, 0)),
        ] + _stats_specs(ci) + _stats_specs(ci) + [
            pl.BlockSpec((co, ci), lambda i: (0, 0)),
            pl.BlockSpec((co, 1), lambda i: (0, 0)),
        ],
        out_specs=[pl.BlockSpec((1, co, m), lambda i: (i, 0, 0))]
        + _stats_specs(co),
        out_shape=[
            jax.ShapeDtypeStruct((b, co, m), jnp.float32),
            jax.ShapeDtypeStruct((co, 1), jnp.float32),
            jax.ShapeDtypeStruct((co, 1), jnp.float32),
        ],
    )(yp, ysum, ysq, gamma.reshape(ci, 1), beta.reshape(ci, 1),
      w, bias.reshape(co, 1))


def _pool(yp, stats, gamma, beta, s, m_count):
    b, c, m = yp.shape
    ysum, ysq = stats
    return pl.pallas_call(
        functools.partial(_pool_body, 1.0 / m_count, s),
        grid=(b,),
        in_specs=[
            pl.BlockSpec((1, c, m), lambda i: (i, 0, 0)),
        ] + _stats_specs(c) + _stats_specs(c),
        out_specs=pl.BlockSpec((1, c, s), lambda i: (i, 0, 0)),
        out_shape=jax.ShapeDtypeStruct((b, c, s), jnp.float32),
    )(yp, ysum, ysq, gamma.reshape(c, 1), beta.reshape(c, 1))


@jax.jit
def kernel(xyz, params):
    l_xyz = [xyz[:, :3, :]]
    l_points = [xyz]
    cur_xyz = xyz[:, :3, :]
    cur_pts = xyz
    for i in range(4):
        s = _NPOINTS[i]
        new_xyz = _fps(cur_xyz, s)
        x = _group(cur_xyz, cur_pts, new_xyz, _RADII[i])
        m_count = _B * _K * s
        layer = params[i]
        w, bias, gamma, beta = layer[0]
        y, ysum, ysq = _conv1(x, w, bias)
        for j in range(1, len(layer)):
            w2, bias2, g2, b2 = layer[j]
            y, ysum2, ysq2 = _conv_mid(
                y, (ysum, ysq), gamma, beta, w2, bias2, m_count)
            ysum, ysq = ysum2, ysq2
            gamma, beta = g2, b2
        out = _pool(y, (ysum, ysq), gamma, beta, s, m_count)
        l_xyz.append(new_xyz)
        l_points.append(out)
        cur_xyz = new_xyz
        cur_pts = out
    return tuple(l_xyz) + tuple(l_points)
